# baseline (speedup 1.0000x reference)
import functools

import jax
import jax.numpy as jnp
from jax import lax
from jax.experimental import pallas as pl
from jax.experimental.pallas import tpu as pltpu
from jax.experimental.pallas import tpu_sc as plsc

_N = 10000
_E = 320000
_D = 128
_G = 64
_NC = 2
_NS = 16
_NW = _NC * _NS
_EPW = _E // _NW
_K = 125
_CHUNKS = _EPW // _K
_NPAD = 10240
_RPT = _NPAD // _NS
_CW = 16
_GC = 16

_R = 1000
_NB = _N // _R

@functools.lru_cache(maxsize=None)
def _sc_kernels():
    mesh = plsc.VectorSubcoreMesh(core_axis_name="c", subcore_axis_name="s",
                                  num_cores=_NC, num_subcores=_NS)

    @functools.partial(
        pl.kernel,
        out_type=jax.ShapeDtypeStruct((_NC, _NPAD, _D), jnp.float32),
        mesh=mesh,
        scratch_types=[
            pltpu.VMEM((2, _GC, _K), jnp.int32),
            pltpu.VMEM((2, _GC, _K), jnp.int32),
            pltpu.VMEM((2, _K, _D), jnp.float32),
            pltpu.VMEM_SHARED((_NPAD, _D), jnp.float32),
            pltpu.SemaphoreType.DMA,
            pltpu.SemaphoreType.DMA,
            pltpu.SemaphoreType.DMA,
            pltpu.SemaphoreType.DMA,
        ],
    )
    def sc_segment_sum(h_hbm, src_hbm, dst_hbm, zeros_hbm, out_hbm,
                       src_v, dst_v, rows_v, acc_sh, i0, i1, g0, g1):
        cid = lax.axis_index("c")
        sid = lax.axis_index("s")
        wid = cid * _NS + sid
        isem = (i0, i1)
        pltpu.sync_copy(zeros_hbm, acc_sh.at[pl.ds(sid * _RPT, _RPT)])
        base = wid * _CHUNKS
        plsc.subcore_barrier()

        ngrp = _CHUNKS // _GC

        def idx_load(g, ibuf):
            gs = pl.ds(base + g * _GC, _GC)
            pltpu.async_copy(src_hbm.at[gs], src_v.at[ibuf], isem[ibuf])
            pltpu.async_copy(dst_hbm.at[gs], dst_v.at[ibuf], isem[ibuf])

        def idx_wait(g, ibuf):
            gs = pl.ds(base + g * _GC, _GC)
            pltpu.make_async_copy(src_hbm.at[gs], src_v.at[ibuf],
                                  isem[ibuf]).wait()
            pltpu.make_async_copy(dst_hbm.at[gs], dst_v.at[ibuf],
                                  isem[ibuf]).wait()

        def gather(b, ibuf, jj, sem):
            pltpu.async_copy(h_hbm.at[src_v.at[ibuf, jj]], rows_v.at[b], sem)

        def gather_wait(b, ibuf, jj, sem):
            pltpu.make_async_copy(h_hbm.at[src_v.at[ibuf, jj]],
                                  rows_v.at[b], sem).wait()

        idx_load(0, 0)
        idx_load(1, 1)
        idx_wait(0, 0)
        gather(0, 0, 0, g0)
        gather(1, 0, 1, g1)

        for g in range(ngrp):
            b = g % 2
            bn = 1 - b

            def body(m, carry):
                jj0 = m * 2
                jj1 = jj0 + 1
                gather_wait(0, b, jj0, g0)
                pltpu.sync_copy(rows_v.at[0], acc_sh.at[dst_v.at[b, jj0]],
                                add=True)
                gather(0, b, jj0 + 2, g0)
                gather_wait(1, b, jj1, g1)
                pltpu.sync_copy(rows_v.at[1], acc_sh.at[dst_v.at[b, jj1]],
                                add=True)
                gather(1, b, jj1 + 2, g1)
                return carry

            lax.fori_loop(0, _GC // 2 - 1, body, 0)

            jl0 = _GC - 2
            jl1 = _GC - 1
            gather_wait(0, b, jl0, g0)
            pltpu.sync_copy(rows_v.at[0], acc_sh.at[dst_v.at[b, jl0]],
                            add=True)
            if g < ngrp - 1:
                idx_wait(g + 1, bn)
                gather(0, bn, 0, g0)
            gather_wait(1, b, jl1, g1)
            pltpu.sync_copy(rows_v.at[1], acc_sh.at[dst_v.at[b, jl1]],
                            add=True)
            if g < ngrp - 1:
                gather(1, bn, 1, g1)
                if g + 2 < ngrp:
                    idx_load(g + 2, b)

        plsc.subcore_barrier()
        pltpu.sync_copy(acc_sh.at[pl.ds(sid * _RPT, _RPT)],
                        out_hbm.at[cid, pl.ds(sid * _RPT, _RPT)])

    @functools.partial(
        pl.kernel,
        out_type=jax.ShapeDtypeStruct((_NC, _NPAD, _D), jnp.float32),
        mesh=mesh,
        scratch_types=[
            pltpu.VMEM((_GC, _K), jnp.int32),
            pltpu.VMEM((_K, _D), jnp.float32),
            pltpu.VMEM_SHARED((_NPAD, _D), jnp.float32),
        ],
    )
    def sc_degree(dst_hbm, ones_hbm, zeros_hbm, out_hbm,
                  dst_v, ones_v, acc_sh):
        cid = lax.axis_index("c")
        sid = lax.axis_index("s")
        wid = cid * _NS + sid
        pltpu.sync_copy(zeros_hbm, acc_sh.at[pl.ds(sid * _RPT, _RPT)])
        pltpu.sync_copy(ones_hbm, ones_v)
        base = wid * _CHUNKS
        plsc.subcore_barrier()

        def group(g, carry):
            pltpu.sync_copy(dst_hbm.at[pl.ds(base + g * _GC, _GC)], dst_v)

            def body(j, carry2):
                pltpu.sync_copy(ones_v, acc_sh.at[dst_v.at[j]], add=True)
                return carry2

            lax.fori_loop(0, _GC, body, carry)
            return carry

        lax.fori_loop(0, _CHUNKS // _GC, group, 0)
        plsc.subcore_barrier()
        pltpu.sync_copy(acc_sh.at[pl.ds(sid * _RPT, _RPT)],
                        out_hbm.at[cid, pl.ds(sid * _RPT, _RPT)])

    return sc_segment_sum, sc_degree


def _hr_body(h, wrT, bl, o_ref):
    o_ref[...] = (jnp.dot(h[...], wrT[...], preferred_element_type=jnp.float32)
                  + bl[...])


def _tc_hr(h, wrT, bl):
    return pl.pallas_call(
        _hr_body,
        grid=(_NB,),
        in_specs=[
            pl.BlockSpec((_R, _D), lambda i: (i, 0)),
            pl.BlockSpec((_D, _D), lambda i: (0, 0)),
            pl.BlockSpec((1, _D), lambda i: (0, 0)),
        ],
        out_specs=pl.BlockSpec((_R, _D), lambda i: (i, 0)),
        out_shape=jax.ShapeDtypeStruct((_N, _D), jnp.float32),
    )(h, wrT, bl)


def _conv_body(s0, s1, c0, c1, hr, wlT, t_ref, st_ref):
    i = pl.program_id(0)
    s = s0[...] + s1[...]
    cnt = c0[...] + c1[...]
    inv = 1.0 / jnp.maximum(cnt, 1.0)
    t = (inv * jnp.dot(s, wlT[...], preferred_element_type=jnp.float32)
         + hr[...])
    t_ref[...] = t

    @pl.when(i == 0)
    def _():
        st_ref[...] = jnp.zeros((8, _D), jnp.float32)

    upd = jnp.concatenate(
        [jnp.sum(t, axis=0)[None, :], jnp.sum(t * t, axis=0)[None, :],
         jnp.zeros((6, _D), jnp.float32)], axis=0)
    st_ref[...] += upd


def _tc_conv(s0, s1, c0, c1, hr, wlT):
    return pl.pallas_call(
        _conv_body,
        grid=(_NB,),
        in_specs=[
            pl.BlockSpec((_R, _D), lambda i: (i, 0)),
            pl.BlockSpec((_R, _D), lambda i: (i, 0)),
            pl.BlockSpec((_R, 1), lambda i: (i, 0)),
            pl.BlockSpec((_R, 1), lambda i: (i, 0)),
            pl.BlockSpec((_R, _D), lambda i: (i, 0)),
            pl.BlockSpec((_D, _D), lambda i: (0, 0)),
        ],
        out_specs=[
            pl.BlockSpec((_R, _D), lambda i: (i, 0)),
            pl.BlockSpec((8, _D), lambda i: (0, 0)),
        ],
        out_shape=[
            jax.ShapeDtypeStruct((_N, _D), jnp.float32),
            jax.ShapeDtypeStruct((8, _D), jnp.float32),
        ],
    )(s0, s1, c0, c1, hr, wlT)


def _bn_body(t, st, gamma, beta, o_ref):
    stt = st[...]
    mu = stt[0:1, :] * (1.0 / _N)
    var = stt[1:2, :] * (1.0 / _N) - mu * mu
    scale = gamma[...] / jnp.sqrt(var + 1e-5)
    shift = beta[...] - mu * scale
    o_ref[...] = jnp.maximum(t[...] * scale + shift, 0.0)


def _tc_bn_relu(t, st, gamma, beta):
    return pl.pallas_call(
        _bn_body,
        grid=(_NB,),
        in_specs=[
            pl.BlockSpec((_R, _D), lambda i: (i, 0)),
            pl.BlockSpec((8, _D), lambda i: (0, 0)),
            pl.BlockSpec((1, _D), lambda i: (0, 0)),
            pl.BlockSpec((1, _D), lambda i: (0, 0)),
        ],
        out_specs=pl.BlockSpec((_R, _D), lambda i: (i, 0)),
        out_shape=jax.ShapeDtypeStruct((_N, _D), jnp.float32),
    )(t, st, gamma, beta)


def _pool_body(s0, s1, c0, c1, hr, wlT, batchb, o_ref, acc_s, acc_c):
    i = pl.program_id(0)
    s = s0[...] + s1[...]
    cnt = c0[...] + c1[...]
    inv = 1.0 / jnp.maximum(cnt, 1.0)
    t = (inv * jnp.dot(s, wlT[...], preferred_element_type=jnp.float32)
         + hr[...])
    b = batchb[...].reshape(_R)
    mask_t = (lax.broadcasted_iota(jnp.int32, (_G, _R), 0)
              == b[None, :]).astype(jnp.float32)

    @pl.when(i == 0)
    def _():
        acc_s[...] = jnp.zeros((_G, _D), jnp.float32)
        acc_c[...] = jnp.zeros((_G, _D), jnp.float32)

    acc_s[...] += jnp.dot(mask_t, t, preferred_element_type=jnp.float32)
    acc_c[...] += jnp.dot(mask_t, jnp.ones((_R, _D), jnp.float32),
                          preferred_element_type=jnp.float32)

    @pl.when(i == _NB - 1)
    def _():
        o_ref[...] = acc_s[...] / jnp.maximum(acc_c[...], 1.0)


def _tc_conv_pool(s0, s1, c0, c1, hr, wlT, batch3):
    return pl.pallas_call(
        _pool_body,
        grid=(_NB,),
        in_specs=[
            pl.BlockSpec((_R, _D), lambda i: (i, 0)),
            pl.BlockSpec((_R, _D), lambda i: (i, 0)),
            pl.BlockSpec((_R, 1), lambda i: (i, 0)),
            pl.BlockSpec((_R, 1), lambda i: (i, 0)),
            pl.BlockSpec((_R, _D), lambda i: (i, 0)),
            pl.BlockSpec((_D, _D), lambda i: (0, 0)),
            pl.BlockSpec((1, 1, _R), lambda i: (i, 0, 0)),
        ],
        out_specs=pl.BlockSpec((_G, _D), lambda i: (0, 0)),
        out_shape=jax.ShapeDtypeStruct((_G, _D), jnp.float32),
        scratch_shapes=[
            pltpu.VMEM((_G, _D), jnp.float32),
            pltpu.VMEM((_G, _D), jnp.float32),
        ],
    )(s0, s1, c0, c1, hr, wlT, batch3)


def kernel(x, edge_index, batch, params):
    src = edge_index[0].reshape(_E // _K, _K)
    dst = edge_index[1].reshape(_E // _K, _K)
    batch3 = batch.reshape(_NB, 1, _R)
    zeros_rows = jnp.zeros((_RPT, _D), jnp.float32)
    ones_rows = jnp.ones((_K, _D), jnp.float32)

    sc_segment_sum, sc_degree = _sc_kernels()
    cnt2 = sc_degree(dst, ones_rows, zeros_rows)
    c0 = cnt2[0, :_N, :1]
    c1 = cnt2[1, :_N, :1]

    h = x
    for li, layer in enumerate(params):
        wlT = layer['Wl'].T
        wrT = layer['Wr'].T
        bl = layer['bl'].reshape(1, _D)
        hr = _tc_hr(h, wrT, bl)
        s2 = sc_segment_sum(h, src, dst, zeros_rows)
        s0, s1 = s2[0], s2[1]
        if li < len(params) - 1:
            t, st = _tc_conv(s0, s1, c0, c1, hr, wlT)
            h = _tc_bn_relu(t, st, layer['gamma'].reshape(1, _D),
                            layer['beta'].reshape(1, _D))
        else:
            h = _tc_conv_pool(s0, s1, c0, c1, hr, wlT, batch3)
    return h


# speedup vs baseline: 1.0093x; 1.0093x over previous
import functools

import jax
import jax.numpy as jnp
from jax import lax
from jax.experimental import pallas as pl
from jax.experimental.pallas import tpu as pltpu
from jax.experimental.pallas import tpu_sc as plsc

_N = 10000
_E = 320000
_D = 128
_G = 64
_NC = 2
_NS = 16
_NW = _NC * _NS
_EPW = _E // _NW
_K = 125
_CHUNKS = _EPW // _K
_NPAD = 10240
_RPT = _NPAD // _NS
_CW = 16
_GC = 16

_R = 1000
_NB = _N // _R

@functools.lru_cache(maxsize=None)
def _sc_kernels():
    mesh = plsc.VectorSubcoreMesh(core_axis_name="c", subcore_axis_name="s",
                                  num_cores=_NC, num_subcores=_NS)

    @functools.partial(
        pl.kernel,
        out_type=jax.ShapeDtypeStruct((_NC, _NPAD, _D), jnp.float32),
        mesh=mesh,
        scratch_types=[
            pltpu.VMEM((2, _GC, _K), jnp.int32),
            pltpu.VMEM((2, _GC, _K), jnp.int32),
            pltpu.VMEM((2, _K, _D), jnp.float32),
            pltpu.VMEM_SHARED((_NPAD, _D), jnp.float32),
            pltpu.SemaphoreType.DMA,
            pltpu.SemaphoreType.DMA,
            pltpu.SemaphoreType.DMA,
            pltpu.SemaphoreType.DMA,
        ],
    )
    def sc_segment_sum(h_hbm, src_hbm, dst_hbm, zeros_hbm, out_hbm,
                       src_v, dst_v, rows_v, acc_sh, i0, i1, g0, g1):
        cid = lax.axis_index("c")
        sid = lax.axis_index("s")
        wid = cid * _NS + sid
        isem = (i0, i1)
        pltpu.sync_copy(zeros_hbm, acc_sh.at[pl.ds(sid * _RPT, _RPT)])
        base = wid * _CHUNKS
        plsc.subcore_barrier()

        ngrp = _CHUNKS // _GC

        def idx_load(g, ibuf):
            gs = pl.ds(base + g * _GC, _GC)
            pltpu.async_copy(src_hbm.at[gs], src_v.at[ibuf], isem[ibuf])
            pltpu.async_copy(dst_hbm.at[gs], dst_v.at[ibuf], isem[ibuf])

        def idx_wait(g, ibuf):
            gs = pl.ds(base + g * _GC, _GC)
            pltpu.make_async_copy(src_hbm.at[gs], src_v.at[ibuf],
                                  isem[ibuf]).wait()
            pltpu.make_async_copy(dst_hbm.at[gs], dst_v.at[ibuf],
                                  isem[ibuf]).wait()

        def gather(b, ibuf, jj, sem):
            pltpu.async_copy(h_hbm.at[src_v.at[ibuf, jj]], rows_v.at[b], sem)

        def gather_wait(b, ibuf, jj, sem):
            pltpu.make_async_copy(h_hbm.at[src_v.at[ibuf, jj]],
                                  rows_v.at[b], sem).wait()

        idx_load(0, 0)
        idx_load(1, 1)
        idx_wait(0, 0)
        gather(0, 0, 0, g0)
        gather(1, 0, 1, g1)

        for g in range(ngrp):
            b = g % 2
            bn = 1 - b

            def body(m, carry):
                jj0 = m * 2
                jj1 = jj0 + 1
                gather_wait(0, b, jj0, g0)
                pltpu.sync_copy(rows_v.at[0], acc_sh.at[dst_v.at[b, jj0]],
                                add=True)
                gather(0, b, jj0 + 2, g0)
                gather_wait(1, b, jj1, g1)
                pltpu.sync_copy(rows_v.at[1], acc_sh.at[dst_v.at[b, jj1]],
                                add=True)
                gather(1, b, jj1 + 2, g1)
                return carry

            lax.fori_loop(0, _GC // 2 - 1, body, 0)

            jl0 = _GC - 2
            jl1 = _GC - 1
            gather_wait(0, b, jl0, g0)
            pltpu.sync_copy(rows_v.at[0], acc_sh.at[dst_v.at[b, jl0]],
                            add=True)
            if g < ngrp - 1:
                idx_wait(g + 1, bn)
                gather(0, bn, 0, g0)
            gather_wait(1, b, jl1, g1)
            pltpu.sync_copy(rows_v.at[1], acc_sh.at[dst_v.at[b, jl1]],
                            add=True)
            if g < ngrp - 1:
                gather(1, bn, 1, g1)
                if g + 2 < ngrp:
                    idx_load(g + 2, b)

        plsc.subcore_barrier()
        pltpu.sync_copy(acc_sh.at[pl.ds(sid * _RPT, _RPT)],
                        out_hbm.at[cid, pl.ds(sid * _RPT, _RPT)])

    @functools.partial(
        pl.kernel,
        out_type=jax.ShapeDtypeStruct((_NC, _NPAD, _D), jnp.float32),
        mesh=mesh,
        scratch_types=[
            pltpu.VMEM((_GC, _K), jnp.int32),
            pltpu.VMEM((_K, _D), jnp.float32),
            pltpu.VMEM_SHARED((_NPAD, _D), jnp.float32),
        ],
    )
    def sc_degree(dst_hbm, ones_hbm, zeros_hbm, out_hbm,
                  dst_v, ones_v, acc_sh):
        cid = lax.axis_index("c")
        sid = lax.axis_index("s")
        wid = cid * _NS + sid
        pltpu.sync_copy(zeros_hbm, acc_sh.at[pl.ds(sid * _RPT, _RPT)])
        pltpu.sync_copy(ones_hbm, ones_v)
        base = wid * _CHUNKS
        plsc.subcore_barrier()

        def group(g, carry):
            pltpu.sync_copy(dst_hbm.at[pl.ds(base + g * _GC, _GC)], dst_v)

            def body(j, carry2):
                pltpu.sync_copy(ones_v, acc_sh.at[dst_v.at[j]], add=True)
                return carry2

            lax.fori_loop(0, _GC, body, carry)
            return carry

        lax.fori_loop(0, _CHUNKS // _GC, group, 0)
        plsc.subcore_barrier()
        pltpu.sync_copy(acc_sh.at[pl.ds(sid * _RPT, _RPT)],
                        out_hbm.at[cid, pl.ds(sid * _RPT, _RPT)])

    return sc_segment_sum, sc_degree


def _conv_body(s0, s1, c0, c1, h, wlT, bl, wrT, t_ref, st_ref):
    i = pl.program_id(0)
    s = s0[...] + s1[...]
    cnt = c0[...] + c1[...]
    inv = 1.0 / jnp.maximum(cnt, 1.0)
    t = (inv * jnp.dot(s, wlT[...], preferred_element_type=jnp.float32)
         + bl[...]
         + jnp.dot(h[...], wrT[...], preferred_element_type=jnp.float32))
    t_ref[...] = t

    @pl.when(i == 0)
    def _():
        st_ref[...] = jnp.zeros((8, _D), jnp.float32)

    upd = jnp.concatenate(
        [jnp.sum(t, axis=0)[None, :], jnp.sum(t * t, axis=0)[None, :],
         jnp.zeros((6, _D), jnp.float32)], axis=0)
    st_ref[...] += upd


def _tc_conv(s0, s1, c0, c1, h, wlT, bl, wrT):
    return pl.pallas_call(
        _conv_body,
        grid=(_NB,),
        in_specs=[
            pl.BlockSpec((_R, _D), lambda i: (i, 0)),
            pl.BlockSpec((_R, _D), lambda i: (i, 0)),
            pl.BlockSpec((_R, 1), lambda i: (i, 0)),
            pl.BlockSpec((_R, 1), lambda i: (i, 0)),
            pl.BlockSpec((_R, _D), lambda i: (i, 0)),
            pl.BlockSpec((_D, _D), lambda i: (0, 0)),
            pl.BlockSpec((1, _D), lambda i: (0, 0)),
            pl.BlockSpec((_D, _D), lambda i: (0, 0)),
        ],
        out_specs=[
            pl.BlockSpec((_R, _D), lambda i: (i, 0)),
            pl.BlockSpec((8, _D), lambda i: (0, 0)),
        ],
        out_shape=[
            jax.ShapeDtypeStruct((_N, _D), jnp.float32),
            jax.ShapeDtypeStruct((8, _D), jnp.float32),
        ],
    )(s0, s1, c0, c1, h, wlT, bl, wrT)


def _bn_body(t, st, gamma, beta, o_ref):
    stt = st[...]
    mu = stt[0:1, :] * (1.0 / _N)
    var = stt[1:2, :] * (1.0 / _N) - mu * mu
    scale = gamma[...] / jnp.sqrt(var + 1e-5)
    shift = beta[...] - mu * scale
    o_ref[...] = jnp.maximum(t[...] * scale + shift, 0.0)


def _tc_bn_relu(t, st, gamma, beta):
    return pl.pallas_call(
        _bn_body,
        grid=(_NB,),
        in_specs=[
            pl.BlockSpec((_R, _D), lambda i: (i, 0)),
            pl.BlockSpec((8, _D), lambda i: (0, 0)),
            pl.BlockSpec((1, _D), lambda i: (0, 0)),
            pl.BlockSpec((1, _D), lambda i: (0, 0)),
        ],
        out_specs=pl.BlockSpec((_R, _D), lambda i: (i, 0)),
        out_shape=jax.ShapeDtypeStruct((_N, _D), jnp.float32),
    )(t, st, gamma, beta)


def _pool_body(s0, s1, c0, c1, h, wlT, bl, wrT, batchb, o_ref, acc_s, acc_c):
    i = pl.program_id(0)
    s = s0[...] + s1[...]
    cnt = c0[...] + c1[...]
    inv = 1.0 / jnp.maximum(cnt, 1.0)
    t = (inv * jnp.dot(s, wlT[...], preferred_element_type=jnp.float32)
         + bl[...]
         + jnp.dot(h[...], wrT[...], preferred_element_type=jnp.float32))
    b = batchb[...].reshape(_R)
    mask_t = (lax.broadcasted_iota(jnp.int32, (_G, _R), 0)
              == b[None, :]).astype(jnp.float32)

    @pl.when(i == 0)
    def _():
        acc_s[...] = jnp.zeros((_G, _D), jnp.float32)
        acc_c[...] = jnp.zeros((_G, _D), jnp.float32)

    acc_s[...] += jnp.dot(mask_t, t, preferred_element_type=jnp.float32)
    acc_c[...] += jnp.dot(mask_t, jnp.ones((_R, _D), jnp.float32),
                          preferred_element_type=jnp.float32)

    @pl.when(i == _NB - 1)
    def _():
        o_ref[...] = acc_s[...] / jnp.maximum(acc_c[...], 1.0)


def _tc_conv_pool(s0, s1, c0, c1, h, wlT, bl, wrT, batch3):
    return pl.pallas_call(
        _pool_body,
        grid=(_NB,),
        in_specs=[
            pl.BlockSpec((_R, _D), lambda i: (i, 0)),
            pl.BlockSpec((_R, _D), lambda i: (i, 0)),
            pl.BlockSpec((_R, 1), lambda i: (i, 0)),
            pl.BlockSpec((_R, 1), lambda i: (i, 0)),
            pl.BlockSpec((_R, _D), lambda i: (i, 0)),
            pl.BlockSpec((_D, _D), lambda i: (0, 0)),
            pl.BlockSpec((1, _D), lambda i: (0, 0)),
            pl.BlockSpec((_D, _D), lambda i: (0, 0)),
            pl.BlockSpec((1, 1, _R), lambda i: (i, 0, 0)),
        ],
        out_specs=pl.BlockSpec((_G, _D), lambda i: (0, 0)),
        out_shape=jax.ShapeDtypeStruct((_G, _D), jnp.float32),
        scratch_shapes=[
            pltpu.VMEM((_G, _D), jnp.float32),
            pltpu.VMEM((_G, _D), jnp.float32),
        ],
    )(s0, s1, c0, c1, h, wlT, bl, wrT, batch3)


def kernel(x, edge_index, batch, params):
    src = edge_index[0].reshape(_E // _K, _K)
    dst = edge_index[1].reshape(_E // _K, _K)
    batch3 = batch.reshape(_NB, 1, _R)
    zeros_rows = jnp.zeros((_RPT, _D), jnp.float32)
    ones_rows = jnp.ones((_K, _D), jnp.float32)

    sc_segment_sum, sc_degree = _sc_kernels()
    cnt2 = sc_degree(dst, ones_rows, zeros_rows)
    c0 = cnt2[0, :_N, :1]
    c1 = cnt2[1, :_N, :1]

    h = x
    for li, layer in enumerate(params):
        wlT = layer['Wl'].T
        wrT = layer['Wr'].T
        bl = layer['bl'].reshape(1, _D)
        s2 = sc_segment_sum(h, src, dst, zeros_rows)
        s0, s1 = s2[0], s2[1]
        if li < len(params) - 1:
            t, st = _tc_conv(s0, s1, c0, c1, h, wlT, bl, wrT)
            h = _tc_bn_relu(t, st, layer['gamma'].reshape(1, _D),
                            layer['beta'].reshape(1, _D))
        else:
            h = _tc_conv_pool(s0, s1, c0, c1, h, wlT, bl, wrT, batch3)
    return h


# speedup vs baseline: 1.0384x; 1.0288x over previous
import functools

import jax
import jax.numpy as jnp
from jax import lax
from jax.experimental import pallas as pl
from jax.experimental.pallas import tpu as pltpu
from jax.experimental.pallas import tpu_sc as plsc

_N = 10000
_E = 320000
_D = 128
_G = 64
_NC = 2
_NS = 16
_NW = _NC * _NS
_EPW = _E // _NW
_K = 125
_CHUNKS = _EPW // _K
_NPAD = 10240
_RPT = _NPAD // _NS
_CW = 16
_GC = 16

_R = 1000
_NB = _N // _R

@functools.lru_cache(maxsize=None)
def _sc_kernels():
    mesh = plsc.VectorSubcoreMesh(core_axis_name="c", subcore_axis_name="s",
                                  num_cores=_NC, num_subcores=_NS)

    @functools.partial(
        pl.kernel,
        out_type=jax.ShapeDtypeStruct((_NC, _NPAD, _D), jnp.float32),
        mesh=mesh,
        scratch_types=[
            pltpu.VMEM((2, _GC, _K), jnp.int32),
            pltpu.VMEM((2, _GC, _K), jnp.int32),
            pltpu.VMEM((2, _K, _D), jnp.float32),
            pltpu.VMEM_SHARED((_NPAD, _D), jnp.float32),
            pltpu.SemaphoreType.DMA,
            pltpu.SemaphoreType.DMA,
            pltpu.SemaphoreType.DMA,
            pltpu.SemaphoreType.DMA,
        ],
    )
    def sc_segment_sum(h_hbm, src_hbm, dst_hbm, zeros_hbm, out_hbm,
                       src_v, dst_v, rows_v, acc_sh, i0, i1, g0, g1):
        cid = lax.axis_index("c")
        sid = lax.axis_index("s")
        wid = cid * _NS + sid
        isem = (i0, i1)
        pltpu.sync_copy(zeros_hbm, acc_sh.at[pl.ds(sid * _RPT, _RPT)])
        base = wid * _CHUNKS
        plsc.subcore_barrier()

        ngrp = _CHUNKS // _GC

        def idx_load(g, ibuf):
            gs = pl.ds(base + g * _GC, _GC)
            pltpu.async_copy(src_hbm.at[gs], src_v.at[ibuf], isem[ibuf])
            pltpu.async_copy(dst_hbm.at[gs], dst_v.at[ibuf], isem[ibuf])

        def idx_wait(g, ibuf):
            gs = pl.ds(base + g * _GC, _GC)
            pltpu.make_async_copy(src_hbm.at[gs], src_v.at[ibuf],
                                  isem[ibuf]).wait()
            pltpu.make_async_copy(dst_hbm.at[gs], dst_v.at[ibuf],
                                  isem[ibuf]).wait()

        def gather(b, ibuf, jj, sem):
            pltpu.async_copy(h_hbm.at[src_v.at[ibuf, jj]], rows_v.at[b], sem)

        def gather_wait(b, ibuf, jj, sem):
            pltpu.make_async_copy(h_hbm.at[src_v.at[ibuf, jj]],
                                  rows_v.at[b], sem).wait()

        idx_load(0, 0)
        idx_load(1, 1)
        idx_wait(0, 0)
        gather(0, 0, 0, g0)
        gather(1, 0, 1, g1)

        for g in range(ngrp):
            b = g % 2
            bn = 1 - b

            def body(m, carry):
                jj0 = m * 2
                jj1 = jj0 + 1
                gather_wait(0, b, jj0, g0)
                pltpu.sync_copy(rows_v.at[0], acc_sh.at[dst_v.at[b, jj0]],
                                add=True)
                gather(0, b, jj0 + 2, g0)
                gather_wait(1, b, jj1, g1)
                pltpu.sync_copy(rows_v.at[1], acc_sh.at[dst_v.at[b, jj1]],
                                add=True)
                gather(1, b, jj1 + 2, g1)
                return carry

            lax.fori_loop(0, _GC // 2 - 1, body, 0)

            jl0 = _GC - 2
            jl1 = _GC - 1
            gather_wait(0, b, jl0, g0)
            pltpu.sync_copy(rows_v.at[0], acc_sh.at[dst_v.at[b, jl0]],
                            add=True)
            if g < ngrp - 1:
                idx_wait(g + 1, bn)
                gather(0, bn, 0, g0)
            gather_wait(1, b, jl1, g1)
            pltpu.sync_copy(rows_v.at[1], acc_sh.at[dst_v.at[b, jl1]],
                            add=True)
            if g < ngrp - 1:
                gather(1, bn, 1, g1)
                if g + 2 < ngrp:
                    idx_load(g + 2, b)

        plsc.subcore_barrier()
        pltpu.sync_copy(acc_sh.at[pl.ds(sid * _RPT, _RPT)],
                        out_hbm.at[cid, pl.ds(sid * _RPT, _RPT)])

    @functools.partial(
        pl.kernel,
        out_type=jax.ShapeDtypeStruct((_NC, _NPAD, _D), jnp.float32),
        mesh=mesh,
        scratch_types=[
            pltpu.VMEM((_GC, _K), jnp.int32),
            pltpu.VMEM((_K, _D), jnp.float32),
            pltpu.VMEM_SHARED((_NPAD, _D), jnp.float32),
        ],
    )
    def sc_degree(dst_hbm, ones_hbm, zeros_hbm, out_hbm,
                  dst_v, ones_v, acc_sh):
        cid = lax.axis_index("c")
        sid = lax.axis_index("s")
        wid = cid * _NS + sid
        pltpu.sync_copy(zeros_hbm, acc_sh.at[pl.ds(sid * _RPT, _RPT)])
        pltpu.sync_copy(ones_hbm, ones_v)
        base = wid * _CHUNKS
        plsc.subcore_barrier()

        def group(g, carry):
            pltpu.sync_copy(dst_hbm.at[pl.ds(base + g * _GC, _GC)], dst_v)

            def body(j, carry2):
                pltpu.sync_copy(ones_v, acc_sh.at[dst_v.at[j]], add=True)
                return carry2

            lax.fori_loop(0, _GC, body, carry)
            return carry

        lax.fori_loop(0, _CHUNKS // _GC, group, 0)
        plsc.subcore_barrier()
        pltpu.sync_copy(acc_sh.at[pl.ds(sid * _RPT, _RPT)],
                        out_hbm.at[cid, pl.ds(sid * _RPT, _RPT)])

    return sc_segment_sum, sc_degree


def _conv_bn_body(s0, s1, c0, c1, h, wlT, bl, wrT, gamma, beta, o_ref,
                  t_s, st_s):
    i = pl.program_id(0)

    @pl.when(i == 0)
    def _():
        st_s[...] = jnp.zeros((8, _D), jnp.float32)

    @pl.when(i < _NB)
    def _():
        s = s0[...] + s1[...]
        cnt = c0[...] + c1[...]
        inv = 1.0 / jnp.maximum(cnt, 1.0)
        t = (inv * jnp.dot(s, wlT[...], preferred_element_type=jnp.float32)
             + bl[...]
             + jnp.dot(h[...], wrT[...], preferred_element_type=jnp.float32))
        t_s[pl.ds(i * _R, _R), :] = t
        upd = jnp.concatenate(
            [jnp.sum(t, axis=0)[None, :], jnp.sum(t * t, axis=0)[None, :],
             jnp.zeros((6, _D), jnp.float32)], axis=0)
        st_s[...] += upd

    @pl.when(i >= _NB)
    def _():
        j = i - _NB
        stt = st_s[...]
        mu = stt[0:1, :] * (1.0 / _N)
        var = stt[1:2, :] * (1.0 / _N) - mu * mu
        scale = gamma[...] / jnp.sqrt(var + 1e-5)
        shift = beta[...] - mu * scale
        t = t_s[pl.ds(j * _R, _R), :]
        o_ref[...] = jnp.maximum(t * scale + shift, 0.0)


def _tc_conv_bn(s0, s1, c0, c1, h, wlT, bl, wrT, gamma, beta):
    phase1 = lambda i: (lax.min(i, _NB - 1), 0)
    return pl.pallas_call(
        _conv_bn_body,
        grid=(2 * _NB,),
        in_specs=[
            pl.BlockSpec((_R, _D), phase1),
            pl.BlockSpec((_R, _D), phase1),
            pl.BlockSpec((_R, 1), phase1),
            pl.BlockSpec((_R, 1), phase1),
            pl.BlockSpec((_R, _D), phase1),
            pl.BlockSpec((_D, _D), lambda i: (0, 0)),
            pl.BlockSpec((1, _D), lambda i: (0, 0)),
            pl.BlockSpec((_D, _D), lambda i: (0, 0)),
            pl.BlockSpec((1, _D), lambda i: (0, 0)),
            pl.BlockSpec((1, _D), lambda i: (0, 0)),
        ],
        out_specs=pl.BlockSpec((_R, _D), lambda i: (lax.max(i - _NB, 0), 0)),
        out_shape=jax.ShapeDtypeStruct((_N, _D), jnp.float32),
        scratch_shapes=[
            pltpu.VMEM((_N, _D), jnp.float32),
            pltpu.VMEM((8, _D), jnp.float32),
        ],
    )(s0, s1, c0, c1, h, wlT, bl, wrT, gamma, beta)


def _pool_body(s0, s1, c0, c1, h, wlT, bl, wrT, batchb, o_ref, acc_s, acc_c):
    i = pl.program_id(0)
    s = s0[...] + s1[...]
    cnt = c0[...] + c1[...]
    inv = 1.0 / jnp.maximum(cnt, 1.0)
    t = (inv * jnp.dot(s, wlT[...], preferred_element_type=jnp.float32)
         + bl[...]
         + jnp.dot(h[...], wrT[...], preferred_element_type=jnp.float32))
    b = batchb[...].reshape(_R)
    mask_t = (lax.broadcasted_iota(jnp.int32, (_G, _R), 0)
              == b[None, :]).astype(jnp.float32)

    @pl.when(i == 0)
    def _():
        acc_s[...] = jnp.zeros((_G, _D), jnp.float32)
        acc_c[...] = jnp.zeros((_G, _D), jnp.float32)

    acc_s[...] += jnp.dot(mask_t, t, preferred_element_type=jnp.float32)
    acc_c[...] += jnp.dot(mask_t, jnp.ones((_R, _D), jnp.float32),
                          preferred_element_type=jnp.float32)

    @pl.when(i == _NB - 1)
    def _():
        o_ref[...] = acc_s[...] / jnp.maximum(acc_c[...], 1.0)


def _tc_conv_pool(s0, s1, c0, c1, h, wlT, bl, wrT, batch3):
    return pl.pallas_call(
        _pool_body,
        grid=(_NB,),
        in_specs=[
            pl.BlockSpec((_R, _D), lambda i: (i, 0)),
            pl.BlockSpec((_R, _D), lambda i: (i, 0)),
            pl.BlockSpec((_R, 1), lambda i: (i, 0)),
            pl.BlockSpec((_R, 1), lambda i: (i, 0)),
            pl.BlockSpec((_R, _D), lambda i: (i, 0)),
            pl.BlockSpec((_D, _D), lambda i: (0, 0)),
            pl.BlockSpec((1, _D), lambda i: (0, 0)),
            pl.BlockSpec((_D, _D), lambda i: (0, 0)),
            pl.BlockSpec((1, 1, _R), lambda i: (i, 0, 0)),
        ],
        out_specs=pl.BlockSpec((_G, _D), lambda i: (0, 0)),
        out_shape=jax.ShapeDtypeStruct((_G, _D), jnp.float32),
        scratch_shapes=[
            pltpu.VMEM((_G, _D), jnp.float32),
            pltpu.VMEM((_G, _D), jnp.float32),
        ],
    )(s0, s1, c0, c1, h, wlT, bl, wrT, batch3)


def kernel(x, edge_index, batch, params):
    src = edge_index[0].reshape(_E // _K, _K)
    dst = edge_index[1].reshape(_E // _K, _K)
    batch3 = batch.reshape(_NB, 1, _R)
    zeros_rows = jnp.zeros((_RPT, _D), jnp.float32)
    ones_rows = jnp.ones((_K, _D), jnp.float32)

    sc_segment_sum, sc_degree = _sc_kernels()
    cnt2 = sc_degree(dst, ones_rows, zeros_rows)
    c0 = cnt2[0, :_N, :1]
    c1 = cnt2[1, :_N, :1]

    h = x
    for li, layer in enumerate(params):
        wlT = layer['Wl'].T
        wrT = layer['Wr'].T
        bl = layer['bl'].reshape(1, _D)
        s2 = sc_segment_sum(h, src, dst, zeros_rows)
        s0, s1 = s2[0], s2[1]
        if li < len(params) - 1:
            h = _tc_conv_bn(s0, s1, c0, c1, h, wlT, bl, wrT,
                            layer['gamma'].reshape(1, _D),
                            layer['beta'].reshape(1, _D))
        else:
            h = _tc_conv_pool(s0, s1, c0, c1, h, wlT, bl, wrT, batch3)
    return h


# speedup vs baseline: 1.0467x; 1.0081x over previous
import functools

import jax
import jax.numpy as jnp
from jax import lax
from jax.experimental import pallas as pl
from jax.experimental.pallas import tpu as pltpu
from jax.experimental.pallas import tpu_sc as plsc

_N = 10000
_E = 320000
_D = 128
_G = 64
_NC = 2
_NS = 16
_NW = _NC * _NS
_EPW = _E // _NW
_K = 125
_CHUNKS = _EPW // _K
_NPAD = 10240
_RPT = _NPAD // _NS
_CW = 16
_GC = 16

_R = 1000
_NB = _N // _R

@functools.lru_cache(maxsize=None)
def _sc_kernels():
    mesh = plsc.VectorSubcoreMesh(core_axis_name="c", subcore_axis_name="s",
                                  num_cores=_NC, num_subcores=_NS)

    @functools.partial(
        pl.kernel,
        out_type=jax.ShapeDtypeStruct((_NC, _NPAD, _D), jnp.float32),
        mesh=mesh,
        scratch_types=[
            pltpu.VMEM((2, _GC, _K), jnp.int32),
            pltpu.VMEM((2, _GC, _K), jnp.int32),
            pltpu.VMEM((2, _K, _D), jnp.float32),
            pltpu.VMEM_SHARED((_NPAD, _D), jnp.float32),
            pltpu.SemaphoreType.DMA,
            pltpu.SemaphoreType.DMA,
            pltpu.SemaphoreType.DMA,
            pltpu.SemaphoreType.DMA,
            pltpu.SemaphoreType.DMA,
        ],
    )
    def sc_segment_sum(h_hbm, src_hbm, dst_hbm, zeros_hbm, out_hbm,
                       src_v, dst_v, rows_v, acc_sh, i0, i1, g0, g1, z0):
        cid = lax.axis_index("c")
        sid = lax.axis_index("s")
        wid = cid * _NS + sid
        isem = (i0, i1)
        pltpu.async_copy(zeros_hbm, acc_sh.at[pl.ds(sid * _RPT, _RPT)], z0)
        base = wid * _CHUNKS

        ngrp = _CHUNKS // _GC

        def idx_load(g, ibuf):
            gs = pl.ds(base + g * _GC, _GC)
            pltpu.async_copy(src_hbm.at[gs], src_v.at[ibuf], isem[ibuf])
            pltpu.async_copy(dst_hbm.at[gs], dst_v.at[ibuf], isem[ibuf])

        def idx_wait(g, ibuf):
            gs = pl.ds(base + g * _GC, _GC)
            pltpu.make_async_copy(src_hbm.at[gs], src_v.at[ibuf],
                                  isem[ibuf]).wait()
            pltpu.make_async_copy(dst_hbm.at[gs], dst_v.at[ibuf],
                                  isem[ibuf]).wait()

        def gather(b, ibuf, jj, sem):
            pltpu.async_copy(h_hbm.at[src_v.at[ibuf, jj]], rows_v.at[b], sem)

        def gather_wait(b, ibuf, jj, sem):
            pltpu.make_async_copy(h_hbm.at[src_v.at[ibuf, jj]],
                                  rows_v.at[b], sem).wait()

        idx_load(0, 0)
        idx_load(1, 1)
        idx_wait(0, 0)
        gather(0, 0, 0, g0)
        gather(1, 0, 1, g1)
        pltpu.make_async_copy(zeros_hbm, acc_sh.at[pl.ds(sid * _RPT, _RPT)],
                              z0).wait()
        plsc.subcore_barrier()

        for g in range(ngrp):
            b = g % 2
            bn = 1 - b

            def body(m, carry):
                jj0 = m * 2
                jj1 = jj0 + 1
                gather_wait(0, b, jj0, g0)
                pltpu.sync_copy(rows_v.at[0], acc_sh.at[dst_v.at[b, jj0]],
                                add=True)
                gather(0, b, jj0 + 2, g0)
                gather_wait(1, b, jj1, g1)
                pltpu.sync_copy(rows_v.at[1], acc_sh.at[dst_v.at[b, jj1]],
                                add=True)
                gather(1, b, jj1 + 2, g1)
                return carry

            lax.fori_loop(0, _GC // 2 - 1, body, 0)

            jl0 = _GC - 2
            jl1 = _GC - 1
            gather_wait(0, b, jl0, g0)
            pltpu.sync_copy(rows_v.at[0], acc_sh.at[dst_v.at[b, jl0]],
                            add=True)
            if g < ngrp - 1:
                idx_wait(g + 1, bn)
                gather(0, bn, 0, g0)
            gather_wait(1, b, jl1, g1)
            pltpu.sync_copy(rows_v.at[1], acc_sh.at[dst_v.at[b, jl1]],
                            add=True)
            if g < ngrp - 1:
                gather(1, bn, 1, g1)
                if g + 2 < ngrp:
                    idx_load(g + 2, b)

        plsc.subcore_barrier()
        pltpu.sync_copy(acc_sh.at[pl.ds(sid * _RPT, _RPT)],
                        out_hbm.at[cid, pl.ds(sid * _RPT, _RPT)])

    @functools.partial(
        pl.kernel,
        out_type=jax.ShapeDtypeStruct((_NC, _NPAD, _D), jnp.float32),
        mesh=mesh,
        scratch_types=[
            pltpu.VMEM((_GC, _K), jnp.int32),
            pltpu.VMEM((_K, _D), jnp.float32),
            pltpu.VMEM_SHARED((_NPAD, _D), jnp.float32),
        ],
    )
    def sc_degree(dst_hbm, ones_hbm, zeros_hbm, out_hbm,
                  dst_v, ones_v, acc_sh):
        cid = lax.axis_index("c")
        sid = lax.axis_index("s")
        wid = cid * _NS + sid
        pltpu.sync_copy(zeros_hbm, acc_sh.at[pl.ds(sid * _RPT, _RPT)])
        pltpu.sync_copy(ones_hbm, ones_v)
        base = wid * _CHUNKS
        plsc.subcore_barrier()

        def group(g, carry):
            pltpu.sync_copy(dst_hbm.at[pl.ds(base + g * _GC, _GC)], dst_v)

            def body(j, carry2):
                pltpu.sync_copy(ones_v, acc_sh.at[dst_v.at[j]], add=True)
                return carry2

            lax.fori_loop(0, _GC, body, carry)
            return carry

        lax.fori_loop(0, _CHUNKS // _GC, group, 0)
        plsc.subcore_barrier()
        pltpu.sync_copy(acc_sh.at[pl.ds(sid * _RPT, _RPT)],
                        out_hbm.at[cid, pl.ds(sid * _RPT, _RPT)])

    return sc_segment_sum, sc_degree


def _conv_bn_body(s0, s1, c0, c1, h, wlT, bl, wrT, gamma, beta, o_ref,
                  t_s, st_s):
    i = pl.program_id(0)

    @pl.when(i == 0)
    def _():
        st_s[...] = jnp.zeros((8, _D), jnp.float32)

    @pl.when(i < _NB)
    def _():
        s = s0[...] + s1[...]
        cnt = c0[...] + c1[...]
        inv = 1.0 / jnp.maximum(cnt, 1.0)
        t = (inv * jnp.dot(s, wlT[...], preferred_element_type=jnp.float32)
             + bl[...]
             + jnp.dot(h[...], wrT[...], preferred_element_type=jnp.float32))
        t_s[pl.ds(i * _R, _R), :] = t
        upd = jnp.concatenate(
            [jnp.sum(t, axis=0)[None, :], jnp.sum(t * t, axis=0)[None, :],
             jnp.zeros((6, _D), jnp.float32)], axis=0)
        st_s[...] += upd

    @pl.when(i >= _NB)
    def _():
        j = i - _NB
        stt = st_s[...]
        mu = stt[0:1, :] * (1.0 / _N)
        var = stt[1:2, :] * (1.0 / _N) - mu * mu
        scale = gamma[...] / jnp.sqrt(var + 1e-5)
        shift = beta[...] - mu * scale
        t = t_s[pl.ds(j * _R, _R), :]
        o_ref[...] = jnp.maximum(t * scale + shift, 0.0)


def _tc_conv_bn(s0, s1, c0, c1, h, wlT, bl, wrT, gamma, beta):
    phase1 = lambda i: (lax.min(i, _NB - 1), 0)
    return pl.pallas_call(
        _conv_bn_body,
        grid=(2 * _NB,),
        in_specs=[
            pl.BlockSpec((_R, _D), phase1),
            pl.BlockSpec((_R, _D), phase1),
            pl.BlockSpec((_R, 1), phase1),
            pl.BlockSpec((_R, 1), phase1),
            pl.BlockSpec((_R, _D), phase1),
            pl.BlockSpec((_D, _D), lambda i: (0, 0)),
            pl.BlockSpec((1, _D), lambda i: (0, 0)),
            pl.BlockSpec((_D, _D), lambda i: (0, 0)),
            pl.BlockSpec((1, _D), lambda i: (0, 0)),
            pl.BlockSpec((1, _D), lambda i: (0, 0)),
        ],
        out_specs=pl.BlockSpec((_R, _D), lambda i: (lax.max(i - _NB, 0), 0)),
        out_shape=jax.ShapeDtypeStruct((_N, _D), jnp.float32),
        scratch_shapes=[
            pltpu.VMEM((_N, _D), jnp.float32),
            pltpu.VMEM((8, _D), jnp.float32),
        ],
    )(s0, s1, c0, c1, h, wlT, bl, wrT, gamma, beta)


def _pool_body(s0, s1, c0, c1, h, wlT, bl, wrT, batchb, o_ref, acc_s, acc_c):
    i = pl.program_id(0)
    s = s0[...] + s1[...]
    cnt = c0[...] + c1[...]
    inv = 1.0 / jnp.maximum(cnt, 1.0)
    t = (inv * jnp.dot(s, wlT[...], preferred_element_type=jnp.float32)
         + bl[...]
         + jnp.dot(h[...], wrT[...], preferred_element_type=jnp.float32))
    b = batchb[...].reshape(_R)
    mask_t = (lax.broadcasted_iota(jnp.int32, (_G, _R), 0)
              == b[None, :]).astype(jnp.float32)

    @pl.when(i == 0)
    def _():
        acc_s[...] = jnp.zeros((_G, _D), jnp.float32)
        acc_c[...] = jnp.zeros((_G, _D), jnp.float32)

    acc_s[...] += jnp.dot(mask_t, t, preferred_element_type=jnp.float32)
    acc_c[...] += jnp.dot(mask_t, jnp.ones((_R, _D), jnp.float32),
                          preferred_element_type=jnp.float32)

    @pl.when(i == _NB - 1)
    def _():
        o_ref[...] = acc_s[...] / jnp.maximum(acc_c[...], 1.0)


def _tc_conv_pool(s0, s1, c0, c1, h, wlT, bl, wrT, batch3):
    return pl.pallas_call(
        _pool_body,
        grid=(_NB,),
        in_specs=[
            pl.BlockSpec((_R, _D), lambda i: (i, 0)),
            pl.BlockSpec((_R, _D), lambda i: (i, 0)),
            pl.BlockSpec((_R, 1), lambda i: (i, 0)),
            pl.BlockSpec((_R, 1), lambda i: (i, 0)),
            pl.BlockSpec((_R, _D), lambda i: (i, 0)),
            pl.BlockSpec((_D, _D), lambda i: (0, 0)),
            pl.BlockSpec((1, _D), lambda i: (0, 0)),
            pl.BlockSpec((_D, _D), lambda i: (0, 0)),
            pl.BlockSpec((1, 1, _R), lambda i: (i, 0, 0)),
        ],
        out_specs=pl.BlockSpec((_G, _D), lambda i: (0, 0)),
        out_shape=jax.ShapeDtypeStruct((_G, _D), jnp.float32),
        scratch_shapes=[
            pltpu.VMEM((_G, _D), jnp.float32),
            pltpu.VMEM((_G, _D), jnp.float32),
        ],
    )(s0, s1, c0, c1, h, wlT, bl, wrT, batch3)


def kernel(x, edge_index, batch, params):
    src = edge_index[0].reshape(_E // _K, _K)
    dst = edge_index[1].reshape(_E // _K, _K)
    batch3 = batch.reshape(_NB, 1, _R)
    zeros_rows = jnp.zeros((_RPT, _D), jnp.float32)
    ones_rows = jnp.ones((_K, _D), jnp.float32)

    sc_segment_sum, sc_degree = _sc_kernels()
    cnt2 = sc_degree(dst, ones_rows, zeros_rows)
    c0 = cnt2[0, :_N, :1]
    c1 = cnt2[1, :_N, :1]

    h = x
    for li, layer in enumerate(params):
        wlT = layer['Wl'].T
        wrT = layer['Wr'].T
        bl = layer['bl'].reshape(1, _D)
        s2 = sc_segment_sum(h, src, dst, zeros_rows)
        s0, s1 = s2[0], s2[1]
        if li < len(params) - 1:
            h = _tc_conv_bn(s0, s1, c0, c1, h, wlT, bl, wrT,
                            layer['gamma'].reshape(1, _D),
                            layer['beta'].reshape(1, _D))
        else:
            h = _tc_conv_pool(s0, s1, c0, c1, h, wlT, bl, wrT, batch3)
    return h


# speedup vs baseline: 1.0515x; 1.0045x over previous
import functools

import jax
import jax.numpy as jnp
from jax import lax
from jax.experimental import pallas as pl
from jax.experimental.pallas import tpu as pltpu
from jax.experimental.pallas import tpu_sc as plsc

_N = 10000
_E = 320000
_D = 128
_G = 64
_NC = 2
_NS = 16
_NW = _NC * _NS
_EPW = _E // _NW
_K = 125
_CHUNKS = _EPW // _K
_NPAD = 10240
_RPT = _NPAD // _NS
_CW = 16
_GC = 16

_R = 1000
_NB = _N // _R

@functools.lru_cache(maxsize=None)
def _sc_kernels():
    mesh = plsc.VectorSubcoreMesh(core_axis_name="c", subcore_axis_name="s",
                                  num_cores=_NC, num_subcores=_NS)

    @functools.partial(
        pl.kernel,
        out_type=jax.ShapeDtypeStruct((_NC, _NPAD, _D), jnp.float32),
        mesh=mesh,
        scratch_types=[
            pltpu.VMEM((2, _GC, _K), jnp.int32),
            pltpu.VMEM((2, _GC, _K), jnp.int32),
            pltpu.VMEM((2, _K, _D), jnp.float32),
            pltpu.VMEM_SHARED((_NPAD, _D), jnp.float32),
            pltpu.SemaphoreType.DMA,
            pltpu.SemaphoreType.DMA,
            pltpu.SemaphoreType.DMA,
            pltpu.SemaphoreType.DMA,
            pltpu.SemaphoreType.DMA,
        ],
    )
    def sc_segment_sum(h_hbm, src_hbm, dst_hbm, zeros_hbm, out_hbm,
                       src_v, dst_v, rows_v, acc_sh, i0, i1, g0, g1, z0):
        cid = lax.axis_index("c")
        sid = lax.axis_index("s")
        wid = cid * _NS + sid
        isem = (i0, i1)
        pltpu.async_copy(zeros_hbm, acc_sh.at[pl.ds(sid * _RPT, _RPT)], z0)
        base = wid * _CHUNKS

        ngrp = _CHUNKS // _GC

        def idx_load(g, ibuf):
            gs = pl.ds(base + g * _GC, _GC)
            pltpu.async_copy(src_hbm.at[gs], src_v.at[ibuf], isem[ibuf])
            pltpu.async_copy(dst_hbm.at[gs], dst_v.at[ibuf], isem[ibuf])

        def idx_wait(g, ibuf):
            gs = pl.ds(base + g * _GC, _GC)
            pltpu.make_async_copy(src_hbm.at[gs], src_v.at[ibuf],
                                  isem[ibuf]).wait()
            pltpu.make_async_copy(dst_hbm.at[gs], dst_v.at[ibuf],
                                  isem[ibuf]).wait()

        def gather(b, ibuf, jj, sem):
            pltpu.async_copy(h_hbm.at[src_v.at[ibuf, jj]], rows_v.at[b], sem)

        def gather_wait(b, ibuf, jj, sem):
            pltpu.make_async_copy(h_hbm.at[src_v.at[ibuf, jj]],
                                  rows_v.at[b], sem).wait()

        idx_load(0, 0)
        idx_load(1, 1)
        idx_wait(0, 0)
        gather(0, 0, 0, g0)
        gather(1, 0, 1, g1)
        pltpu.make_async_copy(zeros_hbm, acc_sh.at[pl.ds(sid * _RPT, _RPT)],
                              z0).wait()
        plsc.subcore_barrier()

        for g in range(ngrp):
            b = g % 2
            bn = 1 - b

            def body(m, carry):
                jj0 = m * 2
                jj1 = jj0 + 1
                gather_wait(0, b, jj0, g0)
                pltpu.sync_copy(rows_v.at[0], acc_sh.at[dst_v.at[b, jj0]],
                                add=True)
                gather(0, b, jj0 + 2, g0)
                gather_wait(1, b, jj1, g1)
                pltpu.sync_copy(rows_v.at[1], acc_sh.at[dst_v.at[b, jj1]],
                                add=True)
                gather(1, b, jj1 + 2, g1)
                return carry

            lax.fori_loop(0, _GC // 2 - 1, body, 0)

            jl0 = _GC - 2
            jl1 = _GC - 1
            gather_wait(0, b, jl0, g0)
            pltpu.sync_copy(rows_v.at[0], acc_sh.at[dst_v.at[b, jl0]],
                            add=True)
            if g < ngrp - 1:
                idx_wait(g + 1, bn)
                gather(0, bn, 0, g0)
            gather_wait(1, b, jl1, g1)
            pltpu.sync_copy(rows_v.at[1], acc_sh.at[dst_v.at[b, jl1]],
                            add=True)
            if g < ngrp - 1:
                gather(1, bn, 1, g1)
                if g + 2 < ngrp:
                    idx_load(g + 2, b)

        plsc.subcore_barrier()
        pltpu.sync_copy(acc_sh.at[pl.ds(sid * _RPT, _RPT)],
                        out_hbm.at[cid, pl.ds(sid * _RPT, _RPT)])

    @functools.partial(
        pl.kernel,
        out_type=jax.ShapeDtypeStruct((_NC, _NPAD, _D), jnp.float32),
        mesh=mesh,
        scratch_types=[
            pltpu.VMEM((2, _GC, _K), jnp.int32),
            pltpu.VMEM((_K, _D), jnp.float32),
            pltpu.VMEM_SHARED((_NPAD, _D), jnp.float32),
            pltpu.SemaphoreType.DMA,
            pltpu.SemaphoreType.DMA,
            pltpu.SemaphoreType.DMA,
        ],
    )
    def sc_degree(dst_hbm, ones_hbm, zeros_hbm, out_hbm,
                  dst_v, ones_v, acc_sh, i0, i1, z0):
        cid = lax.axis_index("c")
        sid = lax.axis_index("s")
        wid = cid * _NS + sid
        isem = (i0, i1)
        base = wid * _CHUNKS
        ngrp = _CHUNKS // _GC

        def idx_load(g, ibuf):
            pltpu.async_copy(dst_hbm.at[pl.ds(base + g * _GC, _GC)],
                             dst_v.at[ibuf], isem[ibuf])

        def idx_wait(g, ibuf):
            pltpu.make_async_copy(dst_hbm.at[pl.ds(base + g * _GC, _GC)],
                                  dst_v.at[ibuf], isem[ibuf]).wait()

        pltpu.async_copy(zeros_hbm, acc_sh.at[pl.ds(sid * _RPT, _RPT)], z0)
        idx_load(0, 0)
        idx_load(1, 1)
        pltpu.sync_copy(ones_hbm, ones_v)
        idx_wait(0, 0)
        pltpu.make_async_copy(zeros_hbm, acc_sh.at[pl.ds(sid * _RPT, _RPT)],
                              z0).wait()
        plsc.subcore_barrier()

        for g in range(ngrp):
            b = g % 2
            if g >= 1:
                idx_wait(g, b)

            def body(j, carry2):
                pltpu.sync_copy(ones_v, acc_sh.at[dst_v.at[b, j]], add=True)
                return carry2

            lax.fori_loop(0, _GC, body, 0)
            if g + 2 < ngrp:
                idx_load(g + 2, b)
        plsc.subcore_barrier()
        pltpu.sync_copy(acc_sh.at[pl.ds(sid * _RPT, _RPT)],
                        out_hbm.at[cid, pl.ds(sid * _RPT, _RPT)])

    return sc_segment_sum, sc_degree


def _conv_bn_body(s0, s1, c0, c1, h, wlT, bl, wrT, gamma, beta, o_ref,
                  t_s, st_s):
    i = pl.program_id(0)

    @pl.when(i == 0)
    def _():
        st_s[...] = jnp.zeros((8, _D), jnp.float32)

    @pl.when(i < _NB)
    def _():
        s = s0[...] + s1[...]
        cnt = c0[...] + c1[...]
        inv = 1.0 / jnp.maximum(cnt, 1.0)
        t = (inv * jnp.dot(s, wlT[...], preferred_element_type=jnp.float32)
             + bl[...]
             + jnp.dot(h[...], wrT[...], preferred_element_type=jnp.float32))
        t_s[pl.ds(i * _R, _R), :] = t
        upd = jnp.concatenate(
            [jnp.sum(t, axis=0)[None, :], jnp.sum(t * t, axis=0)[None, :],
             jnp.zeros((6, _D), jnp.float32)], axis=0)
        st_s[...] += upd

    @pl.when(i >= _NB)
    def _():
        j = i - _NB
        stt = st_s[...]
        mu = stt[0:1, :] * (1.0 / _N)
        var = stt[1:2, :] * (1.0 / _N) - mu * mu
        scale = gamma[...] / jnp.sqrt(var + 1e-5)
        shift = beta[...] - mu * scale
        t = t_s[pl.ds(j * _R, _R), :]
        o_ref[...] = jnp.maximum(t * scale + shift, 0.0)


def _tc_conv_bn(s0, s1, c0, c1, h, wlT, bl, wrT, gamma, beta):
    phase1 = lambda i: (lax.min(i, _NB - 1), 0)
    return pl.pallas_call(
        _conv_bn_body,
        grid=(2 * _NB,),
        in_specs=[
            pl.BlockSpec((_R, _D), phase1),
            pl.BlockSpec((_R, _D), phase1),
            pl.BlockSpec((_R, 1), phase1),
            pl.BlockSpec((_R, 1), phase1),
            pl.BlockSpec((_R, _D), phase1),
            pl.BlockSpec((_D, _D), lambda i: (0, 0)),
            pl.BlockSpec((1, _D), lambda i: (0, 0)),
            pl.BlockSpec((_D, _D), lambda i: (0, 0)),
            pl.BlockSpec((1, _D), lambda i: (0, 0)),
            pl.BlockSpec((1, _D), lambda i: (0, 0)),
        ],
        out_specs=pl.BlockSpec((_R, _D), lambda i: (lax.max(i - _NB, 0), 0)),
        out_shape=jax.ShapeDtypeStruct((_N, _D), jnp.float32),
        scratch_shapes=[
            pltpu.VMEM((_N, _D), jnp.float32),
            pltpu.VMEM((8, _D), jnp.float32),
        ],
    )(s0, s1, c0, c1, h, wlT, bl, wrT, gamma, beta)


def _pool_body(s0, s1, c0, c1, h, wlT, bl, wrT, batchb, o_ref, acc_s, acc_c):
    i = pl.program_id(0)
    s = s0[...] + s1[...]
    cnt = c0[...] + c1[...]
    inv = 1.0 / jnp.maximum(cnt, 1.0)
    t = (inv * jnp.dot(s, wlT[...], preferred_element_type=jnp.float32)
         + bl[...]
         + jnp.dot(h[...], wrT[...], preferred_element_type=jnp.float32))
    b = batchb[...].reshape(_R)
    mask_t = (lax.broadcasted_iota(jnp.int32, (_G, _R), 0)
              == b[None, :]).astype(jnp.float32)

    @pl.when(i == 0)
    def _():
        acc_s[...] = jnp.zeros((_G, _D), jnp.float32)
        acc_c[...] = jnp.zeros((_G, _D), jnp.float32)

    acc_s[...] += jnp.dot(mask_t, t, preferred_element_type=jnp.float32)
    acc_c[...] += jnp.dot(mask_t, jnp.ones((_R, _D), jnp.float32),
                          preferred_element_type=jnp.float32)

    @pl.when(i == _NB - 1)
    def _():
        o_ref[...] = acc_s[...] / jnp.maximum(acc_c[...], 1.0)


def _tc_conv_pool(s0, s1, c0, c1, h, wlT, bl, wrT, batch3):
    return pl.pallas_call(
        _pool_body,
        grid=(_NB,),
        in_specs=[
            pl.BlockSpec((_R, _D), lambda i: (i, 0)),
            pl.BlockSpec((_R, _D), lambda i: (i, 0)),
            pl.BlockSpec((_R, 1), lambda i: (i, 0)),
            pl.BlockSpec((_R, 1), lambda i: (i, 0)),
            pl.BlockSpec((_R, _D), lambda i: (i, 0)),
            pl.BlockSpec((_D, _D), lambda i: (0, 0)),
            pl.BlockSpec((1, _D), lambda i: (0, 0)),
            pl.BlockSpec((_D, _D), lambda i: (0, 0)),
            pl.BlockSpec((1, 1, _R), lambda i: (i, 0, 0)),
        ],
        out_specs=pl.BlockSpec((_G, _D), lambda i: (0, 0)),
        out_shape=jax.ShapeDtypeStruct((_G, _D), jnp.float32),
        scratch_shapes=[
            pltpu.VMEM((_G, _D), jnp.float32),
            pltpu.VMEM((_G, _D), jnp.float32),
        ],
    )(s0, s1, c0, c1, h, wlT, bl, wrT, batch3)


def kernel(x, edge_index, batch, params):
    src = edge_index[0].reshape(_E // _K, _K)
    dst = edge_index[1].reshape(_E // _K, _K)
    batch3 = batch.reshape(_NB, 1, _R)
    zeros_rows = jnp.zeros((_RPT, _D), jnp.float32)
    ones_rows = jnp.ones((_K, _D), jnp.float32)

    sc_segment_sum, sc_degree = _sc_kernels()
    cnt2 = sc_degree(dst, ones_rows, zeros_rows)
    c0 = cnt2[0, :_N, :1]
    c1 = cnt2[1, :_N, :1]

    h = x
    for li, layer in enumerate(params):
        wlT = layer['Wl'].T
        wrT = layer['Wr'].T
        bl = layer['bl'].reshape(1, _D)
        s2 = sc_segment_sum(h, src, dst, zeros_rows)
        s0, s1 = s2[0], s2[1]
        if li < len(params) - 1:
            h = _tc_conv_bn(s0, s1, c0, c1, h, wlT, bl, wrT,
                            layer['gamma'].reshape(1, _D),
                            layer['beta'].reshape(1, _D))
        else:
            h = _tc_conv_pool(s0, s1, c0, c1, h, wlT, bl, wrT, batch3)
    return h
